# Initial kernel scaffold; baseline (speedup 1.0000x reference)
import functools

import jax
import jax.numpy as jnp
from jax import lax
from jax.experimental import pallas as pl
from jax.experimental.pallas import tpu as pltpu
from jax.experimental.pallas import tpu_sc as plsc

L = 16


def _sc_topk_gather(dists, featsflat, maski, B, N, D, K):
    info = plsc.get_sparse_core_info()
    NC = info.num_cores
    NW = NC * info.num_subcores
    RPW = B // NW
    NB = N // L
    KB = K // L

    mesh = plsc.VectorSubcoreMesh(core_axis_name="c", subcore_axis_name="s")

    @functools.partial(
        pl.kernel,
        mesh=mesh,
        out_type=[
            jax.ShapeDtypeStruct((B, K, D), jnp.float32),
            jax.ShapeDtypeStruct((B, K), jnp.int32),
        ],
        scratch_types=[
            pltpu.VMEM((N,), jnp.float32),
            pltpu.VMEM((N,), jnp.int32),
            pltpu.VMEM((K,), jnp.float32),
            pltpu.VMEM((L,), jnp.float32),
            pltpu.VMEM((K,), jnp.int32),
            pltpu.VMEM((K,), jnp.int32),
            pltpu.VMEM((K,), jnp.int32),
            pltpu.VMEM((K, D), jnp.float32),
            pltpu.SemaphoreType.DMA,
        ],
    )
    def sc_fn(dists_hbm, feats_hbm, mask_hbm, outf_hbm, outm_hbm,
              drow, mrow, best, tref, idxl, idxg, mout, rows, sem):
        cid = lax.axis_index("c")
        sid = lax.axis_index("s")
        wid = sid * NC + cid

        def row_body(r, _):
            row = wid * RPW + r
            pltpu.sync_copy(dists_hbm.at[row], drow)
            pltpu.sync_copy(mask_hbm.at[row], mrow)

            inf = jnp.full((L,), jnp.inf, jnp.float32)
            for j in range(KB):
                best[pl.ds(j * L, L)] = inf
            tref[...] = inf

            def pass_a(i, _):
                d = drow[pl.ds(i * L, L)]
                cnt = jnp.sum((d < tref[...]).astype(jnp.int32))

                @pl.when(cnt > 0)
                def _():
                    ns = lax.rev(lax.sort(d), (0,))
                    b0 = best[pl.ds(0 * L, L)]
                    b1 = best[pl.ds(1 * L, L)]
                    b2 = best[pl.ds(2 * L, L)]
                    l0 = jnp.minimum(b0, b2)
                    h0 = jnp.maximum(b0, b2)
                    l1 = jnp.minimum(b1, ns)
                    h1 = jnp.maximum(b1, ns)
                    a0 = jnp.minimum(l0, l1)
                    a1 = jnp.maximum(l0, l1)
                    a2 = jnp.minimum(h0, h1)
                    nb2 = lax.sort(a2)
                    best[pl.ds(0 * L, L)] = lax.sort(a0)
                    best[pl.ds(1 * L, L)] = lax.sort(a1)
                    best[pl.ds(2 * L, L)] = nb2
                    tref[...] = jnp.full((L,), jnp.max(nb2), jnp.float32)

                return None
            lax.fori_loop(0, NB, pass_a, None)

            tv = tref[...]

            def pass_b1(i, acc):
                d = drow[pl.ds(i * L, L)]
                return acc + (d < tv).astype(jnp.int32)
            c_less = jnp.sum(
                lax.fori_loop(0, NB, pass_b1, jnp.zeros((L,), jnp.int32)))
            m = K - c_less

            def pass_b2(i, carry):
                cw, ct = carry
                d = drow[pl.ds(i * L, L)]
                lt = d < tv
                eq = d == tv
                eqi = eq.astype(jnp.int32)
                ranks = ct + lax.cumsum(eqi) - eqi
                sel = lt | (eq & (ranks < m))
                seli = sel.astype(jnp.int32)
                pos = cw + lax.cumsum(seli) - seli
                gl = i * L + lax.iota(jnp.int32, L)
                plsc.store_scatter(idxl, [pos], gl, mask=sel)
                plsc.store_scatter(idxg, [pos], gl + row * N, mask=sel)
                return cw + jnp.sum(seli), ct + jnp.sum(eqi)
            lax.fori_loop(0, NB, pass_b2,
                          (jnp.int32(0), jnp.int32(0)))

            for j in range(KB):
                iv = idxl[pl.ds(j * L, L)]
                mout[pl.ds(j * L, L)] = plsc.load_gather(mrow, [iv])
            pltpu.async_copy(feats_hbm.at[idxg], rows, sem).wait()
            pltpu.sync_copy(rows, outf_hbm.at[row])
            pltpu.sync_copy(mout, outm_hbm.at[row])
            return ()

        lax.fori_loop(0, RPW, row_body, ())

    return sc_fn(dists, featsflat, maski)


def kernel(dists, feats, coord_mask):
    B, N = dists.shape
    D = feats.shape[2]
    K = min(48, N)
    featsflat = feats.reshape(B * N, D)
    maski = coord_mask.astype(jnp.int32)
    outf, outm = _sc_topk_gather(dists, featsflat, maski, B, N, D, K)
    return outf, outm != 0


# speedup vs baseline: 1.2716x; 1.2716x over previous
import functools

import jax
import jax.numpy as jnp
from jax import lax
from jax.experimental import pallas as pl
from jax.experimental.pallas import tpu as pltpu
from jax.experimental.pallas import tpu_sc as plsc

L = 16


def _sc_topk_gather(dists, featsflat, maski, B, N, D, K):
    info = plsc.get_sparse_core_info()
    NC = info.num_cores
    NW = NC * info.num_subcores
    RPW = B // NW
    NB = N // L
    KB = K // L

    mesh = plsc.VectorSubcoreMesh(core_axis_name="c", subcore_axis_name="s")

    @functools.partial(
        pl.kernel,
        mesh=mesh,
        out_type=[
            jax.ShapeDtypeStruct((B, K, D), jnp.float32),
            jax.ShapeDtypeStruct((B, K), jnp.int32),
        ],
        scratch_types=[
            pltpu.VMEM((N,), jnp.float32),
            pltpu.VMEM((N,), jnp.int32),
            pltpu.VMEM((K,), jnp.float32),
            pltpu.VMEM((L,), jnp.float32),
            pltpu.VMEM((K,), jnp.int32),
            pltpu.VMEM((K,), jnp.int32),
            pltpu.VMEM((K,), jnp.int32),
            pltpu.VMEM((K, D), jnp.float32),
            pltpu.SemaphoreType.DMA,
        ],
        compiler_params=pltpu.CompilerParams(needs_layout_passes=False),
    )
    def sc_fn(dists_hbm, feats_hbm, mask_hbm, outf_hbm, outm_hbm,
              drow, mrow, best, tref, idxl, idxg, mout, rows, sem):
        cid = lax.axis_index("c")
        sid = lax.axis_index("s")
        wid = sid * NC + cid

        def row_body(r, _):
            row = wid * RPW + r
            pltpu.sync_copy(dists_hbm.at[row], drow)
            pltpu.sync_copy(mask_hbm.at[row], mrow)

            inf = jnp.full((L,), jnp.inf, jnp.float32)
            for j in range(KB):
                best[pl.ds(j * L, L)] = inf
            tref[...] = inf

            def pass_a(i, _):
                d = drow[pl.ds(i * L, L)]
                cnt = jnp.sum((d < tref[...]).astype(jnp.int32))

                @pl.when(cnt > 0)
                def _():
                    ns = lax.rev(lax.sort(d), (0,))
                    b0 = best[pl.ds(0 * L, L)]
                    b1 = best[pl.ds(1 * L, L)]
                    b2 = best[pl.ds(2 * L, L)]
                    l0 = jnp.minimum(b0, b2)
                    h0 = jnp.maximum(b0, b2)
                    l1 = jnp.minimum(b1, ns)
                    h1 = jnp.maximum(b1, ns)
                    a0 = jnp.minimum(l0, l1)
                    a1 = jnp.maximum(l0, l1)
                    a2 = jnp.minimum(h0, h1)
                    nb2 = lax.sort(a2)
                    best[pl.ds(0 * L, L)] = lax.sort(a0)
                    best[pl.ds(1 * L, L)] = lax.sort(a1)
                    best[pl.ds(2 * L, L)] = nb2
                    tref[...] = jnp.full((L,), jnp.max(nb2), jnp.float32)

                return None
            lax.fori_loop(0, NB, pass_a, None)

            tv = tref[...]

            def pass_b1(i, acc):
                d = drow[pl.ds(i * L, L)]
                return acc + (d < tv).astype(jnp.int32)
            c_less = jnp.sum(
                lax.fori_loop(0, NB, pass_b1, jnp.zeros((L,), jnp.int32)))
            m = K - c_less

            def pass_b2(i, carry):
                cw, ct = carry
                d = drow[pl.ds(i * L, L)]
                lt = d < tv
                eq = d == tv
                eqi = eq.astype(jnp.int32)
                ranks = ct + lax.cumsum(eqi) - eqi
                sel = lt | (eq & (ranks < m))
                seli = sel.astype(jnp.int32)
                pos = cw + lax.cumsum(seli) - seli
                gl = i * L + lax.iota(jnp.int32, L)
                plsc.store_scatter(idxl, [pos], gl, mask=sel)
                plsc.store_scatter(idxg, [pos], gl + row * N, mask=sel)
                return cw + jnp.sum(seli), ct + jnp.sum(eqi)
            lax.fori_loop(0, NB, pass_b2,
                          (jnp.int32(0), jnp.int32(0)))

            for j in range(KB):
                iv = idxl[pl.ds(j * L, L)]
                mout[pl.ds(j * L, L)] = plsc.load_gather(mrow, [iv])
            pltpu.async_copy(feats_hbm.at[idxg], rows, sem).wait()
            pltpu.sync_copy(rows, outf_hbm.at[row])
            pltpu.sync_copy(mout, outm_hbm.at[row])
            return ()

        lax.fori_loop(0, RPW, row_body, ())

    return sc_fn(dists, featsflat, maski)


def kernel(dists, feats, coord_mask):
    B, N = dists.shape
    D = feats.shape[2]
    K = min(48, N)
    featsflat = feats.reshape(B * N, D)
    maski = coord_mask.astype(jnp.int32)
    outf, outm = _sc_topk_gather(dists, featsflat, maski, B, N, D, K)
    return outf, outm != 0


# speedup vs baseline: 1.5449x; 1.2149x over previous
import functools

import jax
import jax.numpy as jnp
from jax import lax
from jax.experimental import pallas as pl
from jax.experimental.pallas import tpu as pltpu
from jax.experimental.pallas import tpu_sc as plsc

L = 16


def _sc_topk_gather(dists, featsflat, maski, B, N, D, K):
    info = plsc.get_sparse_core_info()
    NC = info.num_cores
    NW = NC * info.num_subcores
    RPW = B // NW
    NB = N // L
    KB = K // L
    UA = 4

    mesh = plsc.VectorSubcoreMesh(core_axis_name="c", subcore_axis_name="s")

    @functools.partial(
        pl.kernel,
        mesh=mesh,
        out_type=[
            jax.ShapeDtypeStruct((B, K, D), jnp.float32),
            jax.ShapeDtypeStruct((B, K), jnp.int32),
        ],
        scratch_types=[
            pltpu.VMEM((N,), jnp.float32),
            pltpu.VMEM((N,), jnp.float32),
            pltpu.VMEM((N,), jnp.int32),
            pltpu.VMEM((N,), jnp.int32),
            pltpu.VMEM((K,), jnp.float32),
            pltpu.VMEM((L,), jnp.float32),
            pltpu.VMEM((K,), jnp.int32),
            pltpu.VMEM((K,), jnp.int32),
            pltpu.VMEM((K,), jnp.int32),
            pltpu.VMEM((K,), jnp.int32),
            pltpu.VMEM((K,), jnp.int32),
            pltpu.VMEM((K, D), jnp.float32),
            pltpu.VMEM((K, D), jnp.float32),
            pltpu.SemaphoreType.DMA,
            pltpu.SemaphoreType.DMA,
            pltpu.SemaphoreType.DMA,
            pltpu.SemaphoreType.DMA,
            pltpu.SemaphoreType.DMA,
            pltpu.SemaphoreType.DMA,
        ],
        compiler_params=pltpu.CompilerParams(needs_layout_passes=False),
    )
    def sc_fn(dists_hbm, feats_hbm, mask_hbm, outf_hbm, outm_hbm,
              drow0, drow1, mrow0, mrow1, best, tref, idxl,
              idxg0, idxg1, mb0, mb1, rows0, rows1,
              semd0, semd1, semg0, semg1, semo0, semo1):
        cid = lax.axis_index("c")
        sid = lax.axis_index("s")
        wid = sid * NC + cid

        drow = [drow0, drow1]
        mrow = [mrow0, mrow1]
        idxg = [idxg0, idxg1]
        mb = [mb0, mb1]
        rows = [rows0, rows1]
        semd = [semd0, semd1]
        semg = [semg0, semg1]
        semo = [semo0, semo1]
        cp_in = [None, None]
        cp_g = [None, None]
        cp_o = [None, None]

        def fire_in(r, b):
            row = wid * RPW + r
            c1 = pltpu.async_copy(dists_hbm.at[row], drow[b], semd[b])
            c2 = pltpu.async_copy(mask_hbm.at[row], mrow[b], semd[b])
            cp_in[b] = (c1, c2)

        def compute(r, b):
            row = wid * RPW + r
            db = drow[b]
            inf = jnp.full((L,), jnp.inf, jnp.float32)
            for j in range(KB):
                best[pl.ds(j * L, L)] = inf
            tref[...] = inf

            def merge(d):
                ns = lax.rev(lax.sort(d), (0,))
                b0 = best[pl.ds(0 * L, L)]
                b1 = best[pl.ds(1 * L, L)]
                b2 = best[pl.ds(2 * L, L)]
                l0 = jnp.minimum(b0, b2)
                h0 = jnp.maximum(b0, b2)
                l1 = jnp.minimum(b1, ns)
                h1 = jnp.maximum(b1, ns)
                a0 = jnp.minimum(l0, l1)
                a1 = jnp.maximum(l0, l1)
                a2 = jnp.minimum(h0, h1)
                nb2 = lax.sort(a2)
                best[pl.ds(0 * L, L)] = lax.sort(a0)
                best[pl.ds(1 * L, L)] = lax.sort(a1)
                best[pl.ds(2 * L, L)] = nb2
                tref[...] = jnp.full((L,), jnp.max(nb2), jnp.float32)

            def pass_a(i, _):
                tv = tref[...]
                ds = [db[pl.ds((i * UA + j) * L, L)] for j in range(UA)]
                cs = [(d < tv).astype(jnp.int32) for d in ds]
                tot = cs[0]
                for j in range(1, UA):
                    tot = tot + cs[j]

                @pl.when(jnp.sum(tot) > 0)
                def _():
                    for j in range(UA):
                        @pl.when(jnp.sum(cs[j]) > 0)
                        def _(j=j):
                            merge(ds[j])

                return None
            lax.fori_loop(0, NB // UA, pass_a, None)

            tv = tref[...]

            def pass_b(i, carry):
                cw, accl = carry
                d = db[pl.ds(i * L, L)]
                le = (d <= tv).astype(jnp.int32)
                pos = cw + lax.cumsum(le) - le
                sel = (le > 0) & (pos < K)
                gl = i * L + lax.iota(jnp.int32, L)
                plsc.store_scatter(idxl, [pos], gl, mask=sel)
                plsc.store_scatter(idxg[b], [pos], gl + row * N, mask=sel)
                return cw + jnp.sum(le), accl + (d < tv).astype(jnp.int32)
            c_le, accl = lax.fori_loop(
                0, NB, pass_b, (jnp.int32(0), jnp.zeros((L,), jnp.int32)),
                unroll=2)
            c_less = jnp.sum(accl)

            @pl.when(c_le != K)
            def _():
                m = K - c_less

                def fb(i, carry):
                    cw, ct = carry
                    d = db[pl.ds(i * L, L)]
                    lt = d < tv
                    eq = d == tv
                    eqi = eq.astype(jnp.int32)
                    ranks = ct + lax.cumsum(eqi) - eqi
                    sel = lt | (eq & (ranks < m))
                    seli = sel.astype(jnp.int32)
                    pos = cw + lax.cumsum(seli) - seli
                    gl = i * L + lax.iota(jnp.int32, L)
                    plsc.store_scatter(idxl, [pos], gl, mask=sel)
                    plsc.store_scatter(idxg[b], [pos], gl + row * N, mask=sel)
                    return cw + jnp.sum(seli), ct + jnp.sum(eqi)
                lax.fori_loop(0, NB, fb, (jnp.int32(0), jnp.int32(0)))

            for j in range(KB):
                iv = idxl[pl.ds(j * L, L)]
                mb[b][pl.ds(j * L, L)] = plsc.load_gather(mrow[b], [iv])

        def fire_gather(b):
            cp_g[b] = pltpu.async_copy(feats_hbm.at[idxg[b]], rows[b], semg[b])

        def fire_out(r, b):
            row = wid * RPW + r
            c1 = pltpu.async_copy(rows[b], outf_hbm.at[row], semo[b])
            c2 = pltpu.async_copy(mb[b], outm_hbm.at[row], semo[b])
            cp_o[b] = (c1, c2)

        fire_in(0, 0)
        for r in range(RPW):
            b = r % 2
            if r + 1 < RPW:
                fire_in(r + 1, 1 - b)
            for c in cp_in[b]:
                c.wait()
            if r >= 2:
                for c in cp_o[b]:
                    c.wait()
            compute(r, b)
            fire_gather(b)
            if r >= 1:
                cp_g[1 - b].wait()
                fire_out(r - 1, 1 - b)
        lb = (RPW - 1) % 2
        cp_g[lb].wait()
        fire_out(RPW - 1, lb)
        for c in cp_o[1 - lb]:
            c.wait()
        for c in cp_o[lb]:
            c.wait()

    return sc_fn(dists, featsflat, maski)


def kernel(dists, feats, coord_mask):
    B, N = dists.shape
    D = feats.shape[2]
    K = min(48, N)
    featsflat = feats.reshape(B * N, D)
    maski = coord_mask.astype(jnp.int32)
    outf, outm = _sc_topk_gather(dists, featsflat, maski, B, N, D, K)
    return outf, outm != 0


# speedup vs baseline: 1.9936x; 1.2904x over previous
import functools

import jax
import jax.numpy as jnp
from jax import lax
from jax.experimental import pallas as pl
from jax.experimental.pallas import tpu as pltpu
from jax.experimental.pallas import tpu_sc as plsc

L = 16
CAP = 256
TAU = 0.05


def _sc_topk_gather(dists, featsflat, maski, B, N, D, K):
    info = plsc.get_sparse_core_info()
    NC = info.num_cores
    NW = NC * info.num_subcores
    RPW = B // NW
    NB = N // L
    KB = K // L
    UA = 4

    mesh = plsc.VectorSubcoreMesh(core_axis_name="c", subcore_axis_name="s")

    @functools.partial(
        pl.kernel,
        mesh=mesh,
        out_type=[
            jax.ShapeDtypeStruct((B, K, D), jnp.float32),
            jax.ShapeDtypeStruct((B, K), jnp.int32),
        ],
        scratch_types=[
            pltpu.VMEM((N,), jnp.float32),
            pltpu.VMEM((N,), jnp.float32),
            pltpu.VMEM((N,), jnp.int32),
            pltpu.VMEM((N,), jnp.int32),
            pltpu.VMEM((K,), jnp.float32),
            pltpu.VMEM((L,), jnp.float32),
            pltpu.VMEM((CAP,), jnp.float32),
            pltpu.VMEM((CAP,), jnp.int32),
            pltpu.VMEM((L,), jnp.int32),
            pltpu.VMEM((L,), jnp.int32),
            pltpu.SMEM((1,), jnp.int32),
            pltpu.VMEM((K,), jnp.int32),
            pltpu.VMEM((K,), jnp.int32),
            pltpu.VMEM((K,), jnp.int32),
            pltpu.VMEM((K,), jnp.int32),
            pltpu.VMEM((K,), jnp.int32),
            pltpu.VMEM((K, D), jnp.float32),
            pltpu.VMEM((K, D), jnp.float32),
            pltpu.SemaphoreType.DMA,
            pltpu.SemaphoreType.DMA,
            pltpu.SemaphoreType.DMA,
            pltpu.SemaphoreType.DMA,
            pltpu.SemaphoreType.DMA,
            pltpu.SemaphoreType.DMA,
        ],
        compiler_params=pltpu.CompilerParams(needs_layout_passes=False),
    )
    def sc_fn(dists_hbm, feats_hbm, mask_hbm, outf_hbm, outm_hbm,
              drow0, drow1, mrow0, mrow1, best, tref, cbuf, ibuf,
              cwref, cw2ref, flag, idxl,
              idxg0, idxg1, mb0, mb1, rows0, rows1,
              semd0, semd1, semg0, semg1, semo0, semo1):
        cid = lax.axis_index("c")
        sid = lax.axis_index("s")
        wid = sid * NC + cid

        drow = [drow0, drow1]
        mrow = [mrow0, mrow1]
        idxg = [idxg0, idxg1]
        mb = [mb0, mb1]
        rows = [rows0, rows1]
        semd = [semd0, semd1]
        semg = [semg0, semg1]
        semo = [semo0, semo1]
        cp_in = [None, None]
        cp_g = [None, None]
        cp_o = [None, None]

        ii = lax.iota(jnp.int32, L)
        zero_i = jnp.zeros((L,), jnp.int32)
        inf = jnp.full((L,), jnp.inf, jnp.float32)
        tauv = jnp.full((L,), TAU, jnp.float32)

        def fire_in(r, b):
            row = wid * RPW + r
            c1 = pltpu.async_copy(dists_hbm.at[row], drow[b], semd[b])
            c2 = pltpu.async_copy(mask_hbm.at[row], mrow[b], semd[b])
            cp_in[b] = (c1, c2)

        def merge(d):
            ns = lax.rev(lax.sort(d), (0,))
            b0 = best[pl.ds(0 * L, L)]
            b1 = best[pl.ds(1 * L, L)]
            b2 = best[pl.ds(2 * L, L)]
            l0 = jnp.minimum(b0, b2)
            h0 = jnp.maximum(b0, b2)
            l1 = jnp.minimum(b1, ns)
            h1 = jnp.maximum(b1, ns)
            a0 = jnp.minimum(l0, l1)
            a1 = jnp.maximum(l0, l1)
            a2 = jnp.minimum(h0, h1)
            nb2 = lax.sort(a2)
            best[pl.ds(0 * L, L)] = lax.sort(a0)
            best[pl.ds(1 * L, L)] = lax.sort(a1)
            best[pl.ds(2 * L, L)] = nb2
            tref[...] = jnp.full((L,), nb2[15], jnp.float32)

        def compute(r, b):
            row = wid * RPW + r
            db = drow[b]
            for j in range(KB):
                best[pl.ds(j * L, L)] = inf
            tref[...] = inf
            for j in range(CAP // L):
                cbuf[pl.ds(j * L, L)] = inf
            cwref[...] = zero_i
            flag[0] = jnp.int32(0)

            def sweep1(i, _):
                d = db[pl.ds(i * L, L)]
                le = d <= tauv
                hit = plsc.all_reduce_population_count(le)

                @pl.when(hit[0] > 0)
                def _():
                    lei = le.astype(jnp.int32)
                    cums = lax.cumsum(lei)
                    cw = cwref[...]
                    pos = cw + cums - lei
                    sel = le & (pos < CAP)
                    gl = i * L + ii
                    plsc.store_scatter(cbuf, [pos], d, mask=sel)
                    plsc.store_scatter(ibuf, [pos], gl, mask=sel)
                    cwref[...] = cw + jnp.full((L,), cums[15], jnp.int32)

                return None
            lax.fori_loop(0, NB, sweep1, None, unroll=2)

            c_tau = cwref[...][0]
            ok1 = (c_tau >= K) & (c_tau <= CAP)

            @pl.when(ok1)
            def _():
                ncb = (c_tau + (L - 1)) // L

                def ca(i, _):
                    merge(cbuf[pl.ds(i * L, L)])
                    return None
                lax.fori_loop(0, ncb, ca, None)

                tv = tref[...]
                cw2ref[...] = zero_i

                def cb(i, _):
                    c = cbuf[pl.ds(i * L, L)]
                    le = c <= tv
                    lei = le.astype(jnp.int32)
                    cums = lax.cumsum(lei)
                    cw = cw2ref[...]
                    pos = cw + cums - lei
                    sel = le & (pos < K)
                    iv = ibuf[pl.ds(i * L, L)]
                    plsc.store_scatter(idxl, [pos], iv, mask=sel)
                    plsc.store_scatter(idxg[b], [pos], iv + row * N, mask=sel)
                    cw2ref[...] = cw + jnp.full((L,), cums[15], jnp.int32)
                    return None
                lax.fori_loop(0, ncb, cb, None)
                flag[0] = (cw2ref[...][0] == K).astype(jnp.int32)

            @pl.when(flag[0] == 0)
            def _():
                for j in range(KB):
                    best[pl.ds(j * L, L)] = inf
                tref[...] = inf

                def pass_a(i, _):
                    tv = tref[...]
                    ds = [db[pl.ds((i * UA + j) * L, L)] for j in range(UA)]
                    ms = [d < tv for d in ds]
                    anyhit = (ms[0] | ms[1]) | (ms[2] | ms[3])
                    cnt = plsc.all_reduce_population_count(anyhit)

                    @pl.when(cnt[0] > 0)
                    def _():
                        for j in range(UA):
                            cj = plsc.all_reduce_population_count(ms[j])

                            @pl.when(cj[0] > 0)
                            def _(j=j):
                                merge(ds[j])

                    return None
                lax.fori_loop(0, NB // UA, pass_a, None)

                tv = tref[...]

                def b1(i, acc):
                    d = db[pl.ds(i * L, L)]
                    return acc + (d < tv).astype(jnp.int32)
                c_less = jnp.sum(
                    lax.fori_loop(0, NB, b1, zero_i))
                m = K - c_less

                def b2(i, carry):
                    cw, ct = carry
                    d = db[pl.ds(i * L, L)]
                    lt = d < tv
                    eq = d == tv
                    eqi = eq.astype(jnp.int32)
                    ranks = ct + lax.cumsum(eqi) - eqi
                    sel = lt | (eq & (ranks < m))
                    seli = sel.astype(jnp.int32)
                    pos = cw + lax.cumsum(seli) - seli
                    gl = i * L + ii
                    plsc.store_scatter(idxl, [pos], gl, mask=sel)
                    plsc.store_scatter(idxg[b], [pos], gl + row * N, mask=sel)
                    return cw + jnp.sum(seli), ct + jnp.sum(eqi)
                lax.fori_loop(0, NB, b2, (jnp.int32(0), jnp.int32(0)))

            for j in range(KB):
                iv = idxl[pl.ds(j * L, L)]
                mb[b][pl.ds(j * L, L)] = plsc.load_gather(mrow[b], [iv])

        def fire_gather(b):
            cp_g[b] = pltpu.async_copy(feats_hbm.at[idxg[b]], rows[b], semg[b])

        def fire_out(r, b):
            row = wid * RPW + r
            c1 = pltpu.async_copy(rows[b], outf_hbm.at[row], semo[b])
            c2 = pltpu.async_copy(mb[b], outm_hbm.at[row], semo[b])
            cp_o[b] = (c1, c2)

        fire_in(0, 0)
        for r in range(RPW):
            b = r % 2
            if r + 1 < RPW:
                fire_in(r + 1, 1 - b)
            for c in cp_in[b]:
                c.wait()
            if r >= 2:
                for c in cp_o[b]:
                    c.wait()
            compute(r, b)
            fire_gather(b)
            if r >= 1:
                cp_g[1 - b].wait()
                fire_out(r - 1, 1 - b)
        lb = (RPW - 1) % 2
        cp_g[lb].wait()
        fire_out(RPW - 1, lb)
        for c in cp_o[1 - lb]:
            c.wait()
        for c in cp_o[lb]:
            c.wait()

    return sc_fn(dists, featsflat, maski)


def kernel(dists, feats, coord_mask):
    B, N = dists.shape
    D = feats.shape[2]
    K = min(48, N)
    featsflat = feats.reshape(B * N, D)
    maski = coord_mask.astype(jnp.int32)
    outf, outm = _sc_topk_gather(dists, featsflat, maski, B, N, D, K)
    return outf, outm != 0


# speedup vs baseline: 2.3900x; 1.1989x over previous
import functools

import jax
import jax.numpy as jnp
from jax import lax
from jax.experimental import pallas as pl
from jax.experimental.pallas import tpu as pltpu
from jax.experimental.pallas import tpu_sc as plsc

L = 16
CAPF = 256
TAU = 0.05


def _sc_topk_gather(dists, featsflat, maskw, B, N, D, K):
    info = plsc.get_sparse_core_info()
    NC = info.num_cores
    NW = NC * info.num_subcores
    RPW = B // NW
    NB = N // L
    KB = K // L
    NW4 = N // 4
    CB = N + L
    UA = 4

    mesh = plsc.VectorSubcoreMesh(core_axis_name="c", subcore_axis_name="s")

    @functools.partial(
        pl.kernel,
        mesh=mesh,
        out_type=[
            jax.ShapeDtypeStruct((B, K, D), jnp.float32),
            jax.ShapeDtypeStruct((B, K), jnp.int32),
        ],
        scratch_types=[
            pltpu.VMEM((N,), jnp.float32),
            pltpu.VMEM((N,), jnp.float32),
            pltpu.VMEM((NW4,), jnp.int32),
            pltpu.VMEM((NW4,), jnp.int32),
            pltpu.VMEM((K,), jnp.float32),
            pltpu.VMEM((L,), jnp.float32),
            pltpu.VMEM((CB,), jnp.float32),
            pltpu.VMEM((CB,), jnp.int32),
            pltpu.VMEM((L,), jnp.int32),
            pltpu.SMEM((1,), jnp.int32),
            pltpu.VMEM((K,), jnp.int32),
            pltpu.VMEM((K,), jnp.int32),
            pltpu.VMEM((K,), jnp.int32),
            pltpu.VMEM((K,), jnp.int32),
            pltpu.VMEM((K,), jnp.int32),
            pltpu.VMEM((K, D), jnp.float32),
            pltpu.VMEM((K, D), jnp.float32),
            pltpu.SemaphoreType.DMA,
            pltpu.SemaphoreType.DMA,
            pltpu.SemaphoreType.DMA,
            pltpu.SemaphoreType.DMA,
            pltpu.SemaphoreType.DMA,
            pltpu.SemaphoreType.DMA,
        ],
        compiler_params=pltpu.CompilerParams(needs_layout_passes=False),
    )
    def sc_fn(dists_hbm, feats_hbm, mask_hbm, outf_hbm, outm_hbm,
              drow0, drow1, mrow0, mrow1, best, tref, cbuf, ibuf,
              cw2ref, flag, idxl,
              idxg0, idxg1, mb0, mb1, rows0, rows1,
              semd0, semd1, semg0, semg1, semo0, semo1):
        cid = lax.axis_index("c")
        sid = lax.axis_index("s")
        wid = sid * NC + cid

        drow = [drow0, drow1]
        mrow = [mrow0, mrow1]
        idxg = [idxg0, idxg1]
        mb = [mb0, mb1]
        rows = [rows0, rows1]
        semd = [semd0, semd1]
        semg = [semg0, semg1]
        semo = [semo0, semo1]
        cp_in = [None, None]
        cp_g = [None, None]
        cp_o = [None, None]

        ii = lax.iota(jnp.int32, L)
        zero_i = jnp.zeros((L,), jnp.int32)
        one_i = jnp.full((L,), 1, jnp.int32)
        inf = jnp.full((L,), jnp.inf, jnp.float32)
        tauv = jnp.full((L,), TAU, jnp.float32)

        def fire_in(r, b):
            row = wid * RPW + r
            c1 = pltpu.async_copy(dists_hbm.at[row], drow[b], semd[b])
            c2 = pltpu.async_copy(mask_hbm.at[row], mrow[b], semd[b])
            cp_in[b] = (c1, c2)

        def merge(d):
            ns = lax.rev(lax.sort(d), (0,))
            b0 = best[pl.ds(0 * L, L)]
            b1 = best[pl.ds(1 * L, L)]
            b2 = best[pl.ds(2 * L, L)]
            l0 = jnp.minimum(b0, b2)
            h0 = jnp.maximum(b0, b2)
            l1 = jnp.minimum(b1, ns)
            h1 = jnp.maximum(b1, ns)
            a0 = jnp.minimum(l0, l1)
            a1 = jnp.maximum(l0, l1)
            a2 = jnp.minimum(h0, h1)
            nb2 = lax.sort(a2)
            best[pl.ds(0 * L, L)] = lax.sort(a0)
            best[pl.ds(1 * L, L)] = lax.sort(a1)
            best[pl.ds(2 * L, L)] = nb2
            tref[...] = jnp.full((L,), nb2[15], jnp.float32)

        def compute(r, b):
            row = wid * RPW + r
            db = drow[b]
            for j in range(KB):
                best[pl.ds(j * L, L)] = inf
            tref[...] = inf
            for j in range(CAPF // L + 1):
                cbuf[pl.ds(j * L, L)] = inf
            flag[0] = jnp.int32(0)

            def sweep1(i, carry):
                cwm1, gl = carry
                d = db[pl.ds(i * L, L)]
                le = d <= tauv
                lei = jnp.where(le, one_i, zero_i)
                cums = lax.cumsum(lei)
                pos = cwm1 + cums
                plsc.store_scatter(cbuf, [pos], d, mask=le)
                plsc.store_scatter(ibuf, [pos], gl, mask=le)
                tot = jnp.full((L,), cums[15], jnp.int32)
                return cwm1 + tot, gl + L
            cwm1, _ = lax.fori_loop(
                0, NB, sweep1, (zero_i - 1, ii), unroll=4)

            c_tau = cwm1[15] + 1
            ok1 = (c_tau >= K) & (c_tau <= CAPF)

            @pl.when(ok1)
            def _():
                ncb = (c_tau + (L - 1)) // L

                def ca(i, _):
                    merge(cbuf[pl.ds(i * L, L)])
                    return None
                lax.fori_loop(0, ncb, ca, None)

                tv = tref[...]
                cw2ref[...] = zero_i - 1

                def cb(i, _):
                    c = cbuf[pl.ds(i * L, L)]
                    le = c <= tv
                    lei = jnp.where(le, one_i, zero_i)
                    cums = lax.cumsum(lei)
                    cwm1c = cw2ref[...]
                    pos = cwm1c + cums
                    sel = le & (pos < K)
                    iv = ibuf[pl.ds(i * L, L)]
                    plsc.store_scatter(idxl, [pos], iv, mask=sel)
                    plsc.store_scatter(idxg[b], [pos], iv + row * N, mask=sel)
                    cw2ref[...] = cwm1c + jnp.full((L,), cums[15], jnp.int32)
                    return None
                lax.fori_loop(0, ncb, cb, None)
                flag[0] = (cw2ref[...][15] + 1 == K).astype(jnp.int32)

            @pl.when(flag[0] == 0)
            def _():
                for j in range(KB):
                    best[pl.ds(j * L, L)] = inf
                tref[...] = inf

                def pass_a(i, _):
                    tv = tref[...]
                    ds = [db[pl.ds((i * UA + j) * L, L)] for j in range(UA)]
                    ms = [d < tv for d in ds]
                    anyhit = (ms[0] | ms[1]) | (ms[2] | ms[3])
                    cnt = plsc.all_reduce_population_count(anyhit)

                    @pl.when(cnt[0] > 0)
                    def _():
                        for j in range(UA):
                            cj = plsc.all_reduce_population_count(ms[j])

                            @pl.when(cj[0] > 0)
                            def _(j=j):
                                merge(ds[j])

                    return None
                lax.fori_loop(0, NB // UA, pass_a, None)

                tv = tref[...]

                def b1(i, acc):
                    d = db[pl.ds(i * L, L)]
                    return acc + (d < tv).astype(jnp.int32)
                c_less = jnp.sum(
                    lax.fori_loop(0, NB, b1, zero_i))
                m = K - c_less

                def b2(i, carry):
                    cw, ct = carry
                    d = db[pl.ds(i * L, L)]
                    lt = d < tv
                    eq = d == tv
                    eqi = eq.astype(jnp.int32)
                    ranks = ct + lax.cumsum(eqi) - eqi
                    sel = lt | (eq & (ranks < m))
                    seli = sel.astype(jnp.int32)
                    pos = cw + lax.cumsum(seli) - seli
                    gl = i * L + ii
                    plsc.store_scatter(idxl, [pos], gl, mask=sel)
                    plsc.store_scatter(idxg[b], [pos], gl + row * N, mask=sel)
                    return cw + jnp.sum(seli), ct + jnp.sum(eqi)
                lax.fori_loop(0, NB, b2, (jnp.int32(0), jnp.int32(0)))

            for j in range(KB):
                iv = idxl[pl.ds(j * L, L)]
                w = plsc.load_gather(mrow[b], [jnp.right_shift(iv, 2)])
                sh = jnp.left_shift(iv & 3, 3)
                mb[b][pl.ds(j * L, L)] = jnp.right_shift(w, sh) & 1

        def fire_gather(b):
            cp_g[b] = pltpu.async_copy(feats_hbm.at[idxg[b]], rows[b], semg[b])

        def fire_out(r, b):
            row = wid * RPW + r
            c1 = pltpu.async_copy(rows[b], outf_hbm.at[row], semo[b])
            c2 = pltpu.async_copy(mb[b], outm_hbm.at[row], semo[b])
            cp_o[b] = (c1, c2)

        fire_in(0, 0)
        for r in range(RPW):
            b = r % 2
            if r + 1 < RPW:
                fire_in(r + 1, 1 - b)
            for c in cp_in[b]:
                c.wait()
            if r >= 2:
                for c in cp_o[b]:
                    c.wait()
            compute(r, b)
            fire_gather(b)
            if r >= 1:
                cp_g[1 - b].wait()
                fire_out(r - 1, 1 - b)
        lb = (RPW - 1) % 2
        cp_g[lb].wait()
        fire_out(RPW - 1, lb)
        for c in cp_o[1 - lb]:
            c.wait()
        for c in cp_o[lb]:
            c.wait()

    return sc_fn(dists, featsflat, maskw)


def kernel(dists, feats, coord_mask):
    B, N = dists.shape
    D = feats.shape[2]
    K = min(48, N)
    featsflat = feats.reshape(B * N, D)
    maskw = lax.bitcast_convert_type(
        coord_mask.astype(jnp.uint8).reshape(B, N // 4, 4), jnp.int32)
    outf, outm = _sc_topk_gather(dists, featsflat, maskw, B, N, D, K)
    return outf, outm != 0


# speedup vs baseline: 2.6104x; 1.0922x over previous
import functools

import jax
import jax.numpy as jnp
from jax import lax
from jax.experimental import pallas as pl
from jax.experimental.pallas import tpu as pltpu
from jax.experimental.pallas import tpu_sc as plsc

L = 16
CAPF = 256
TAU = 0.05


def _sc_topk_gather(dists, featsflat, maskw, B, N, D, K):
    info = plsc.get_sparse_core_info()
    NC = info.num_cores
    NW = NC * info.num_subcores
    RPW = B // NW
    NB = N // L
    KB = K // L
    NW4 = N // 4
    CB = N + L
    UA = 4

    mesh = plsc.VectorSubcoreMesh(core_axis_name="c", subcore_axis_name="s")

    @functools.partial(
        pl.kernel,
        mesh=mesh,
        out_type=[
            jax.ShapeDtypeStruct((B, K, D), jnp.float32),
            jax.ShapeDtypeStruct((B, K), jnp.int32),
        ],
        scratch_types=[
            pltpu.VMEM((N,), jnp.float32),
            pltpu.VMEM((N,), jnp.float32),
            pltpu.VMEM((NW4,), jnp.int32),
            pltpu.VMEM((NW4,), jnp.int32),
            pltpu.VMEM((K,), jnp.float32),
            pltpu.VMEM((L,), jnp.float32),
            pltpu.VMEM((CB,), jnp.float32),
            pltpu.VMEM((CB,), jnp.int32),
            pltpu.VMEM((L,), jnp.int32),
            pltpu.SMEM((1,), jnp.int32),
            pltpu.VMEM((K,), jnp.int32),
            pltpu.VMEM((K,), jnp.int32),
            pltpu.VMEM((K,), jnp.int32),
            pltpu.VMEM((K,), jnp.int32),
            pltpu.VMEM((K,), jnp.int32),
            pltpu.VMEM((K, D), jnp.float32),
            pltpu.VMEM((K, D), jnp.float32),
            pltpu.SemaphoreType.DMA,
            pltpu.SemaphoreType.DMA,
            pltpu.SemaphoreType.DMA,
            pltpu.SemaphoreType.DMA,
            pltpu.SemaphoreType.DMA,
            pltpu.SemaphoreType.DMA,
        ],
        compiler_params=pltpu.CompilerParams(needs_layout_passes=False),
    )
    def sc_fn(dists_hbm, feats_hbm, mask_hbm, outf_hbm, outm_hbm,
              drow0, drow1, mrow0, mrow1, best, tref, cbuf, ibuf,
              cw2ref, flag, idxl,
              idxg0, idxg1, mb0, mb1, rows0, rows1,
              semd0, semd1, semg0, semg1, semo0, semo1):
        cid = lax.axis_index("c")
        sid = lax.axis_index("s")
        wid = sid * NC + cid

        drow = [drow0, drow1]
        mrow = [mrow0, mrow1]
        idxg = [idxg0, idxg1]
        mb = [mb0, mb1]
        rows = [rows0, rows1]
        semd = [semd0, semd1]
        semg = [semg0, semg1]
        semo = [semo0, semo1]
        cp_in = [None, None]
        cp_g = [None, None]
        cp_o = [None, None]

        ii = lax.iota(jnp.int32, L)
        zero_i = jnp.zeros((L,), jnp.int32)
        one_i = jnp.full((L,), 1, jnp.int32)
        inf = jnp.full((L,), jnp.inf, jnp.float32)
        tauv = jnp.full((L,), TAU, jnp.float32)

        def fire_in(r, b):
            row = wid * RPW + r
            c1 = pltpu.async_copy(dists_hbm.at[row], drow[b], semd[b])
            c2 = pltpu.async_copy(mask_hbm.at[row], mrow[b], semd[b])
            cp_in[b] = (c1, c2)

        def merge(d):
            ns = lax.rev(lax.sort(d), (0,))
            b0 = best[pl.ds(0 * L, L)]
            b1 = best[pl.ds(1 * L, L)]
            b2 = best[pl.ds(2 * L, L)]
            l0 = jnp.minimum(b0, b2)
            h0 = jnp.maximum(b0, b2)
            l1 = jnp.minimum(b1, ns)
            h1 = jnp.maximum(b1, ns)
            a0 = jnp.minimum(l0, l1)
            a1 = jnp.maximum(l0, l1)
            a2 = jnp.minimum(h0, h1)
            nb2 = lax.sort(a2)
            best[pl.ds(0 * L, L)] = lax.sort(a0)
            best[pl.ds(1 * L, L)] = lax.sort(a1)
            best[pl.ds(2 * L, L)] = nb2
            tref[...] = jnp.full((L,), nb2[15], jnp.float32)

        def compute(r, b):
            row = wid * RPW + r
            db = drow[b]
            for j in range(KB):
                best[pl.ds(j * L, L)] = inf
            tref[...] = inf
            for j in range(CAPF // L + 1):
                cbuf[pl.ds(j * L, L)] = inf
            flag[0] = jnp.int32(0)

            G = 4

            def sweep1(g, carry):
                cwm1, gl = carry
                ds, les, cums_l = [], [], []
                for j in range(G):
                    d = db[pl.ds((g * G + j) * L, L)]
                    le = d <= tauv
                    lei = jnp.where(le, one_i, zero_i)
                    ds.append(d)
                    les.append(le)
                    cums_l.append(lax.cumsum(lei))
                offs = [cwm1]
                for j in range(G):
                    offs.append(
                        offs[j] + jnp.full((L,), cums_l[j][15], jnp.int32))
                for j in range(G):
                    pos = offs[j] + cums_l[j]
                    plsc.store_scatter(cbuf, [pos], ds[j], mask=les[j])
                    plsc.store_scatter(ibuf, [pos], gl + j * L, mask=les[j])
                return offs[G], gl + G * L
            cwm1, _ = lax.fori_loop(
                0, NB // G, sweep1, (zero_i - 1, ii))

            c_tau = cwm1[15] + 1
            ok1 = (c_tau >= K) & (c_tau <= CAPF)

            @pl.when(ok1)
            def _():
                ncb = (c_tau + (L - 1)) // L

                def ca(i, _):
                    merge(cbuf[pl.ds(i * L, L)])
                    return None
                lax.fori_loop(0, ncb, ca, None)

                tv = tref[...]
                cw2ref[...] = zero_i - 1

                def cb(i, _):
                    c = cbuf[pl.ds(i * L, L)]
                    le = c <= tv
                    lei = jnp.where(le, one_i, zero_i)
                    cums = lax.cumsum(lei)
                    cwm1c = cw2ref[...]
                    pos = cwm1c + cums
                    sel = le & (pos < K)
                    iv = ibuf[pl.ds(i * L, L)]
                    plsc.store_scatter(idxl, [pos], iv, mask=sel)
                    plsc.store_scatter(idxg[b], [pos], iv + row * N, mask=sel)
                    cw2ref[...] = cwm1c + jnp.full((L,), cums[15], jnp.int32)
                    return None
                lax.fori_loop(0, ncb, cb, None)
                flag[0] = (cw2ref[...][15] + 1 == K).astype(jnp.int32)

            @pl.when(flag[0] == 0)
            def _():
                for j in range(KB):
                    best[pl.ds(j * L, L)] = inf
                tref[...] = inf

                def pass_a(i, _):
                    tv = tref[...]
                    ds = [db[pl.ds((i * UA + j) * L, L)] for j in range(UA)]
                    ms = [d < tv for d in ds]
                    anyhit = (ms[0] | ms[1]) | (ms[2] | ms[3])
                    cnt = plsc.all_reduce_population_count(anyhit)

                    @pl.when(cnt[0] > 0)
                    def _():
                        for j in range(UA):
                            cj = plsc.all_reduce_population_count(ms[j])

                            @pl.when(cj[0] > 0)
                            def _(j=j):
                                merge(ds[j])

                    return None
                lax.fori_loop(0, NB // UA, pass_a, None)

                tv = tref[...]

                def b1(i, acc):
                    d = db[pl.ds(i * L, L)]
                    return acc + (d < tv).astype(jnp.int32)
                c_less = jnp.sum(
                    lax.fori_loop(0, NB, b1, zero_i))
                m = K - c_less

                def b2(i, carry):
                    cw, ct = carry
                    d = db[pl.ds(i * L, L)]
                    lt = d < tv
                    eq = d == tv
                    eqi = eq.astype(jnp.int32)
                    ranks = ct + lax.cumsum(eqi) - eqi
                    sel = lt | (eq & (ranks < m))
                    seli = sel.astype(jnp.int32)
                    pos = cw + lax.cumsum(seli) - seli
                    gl = i * L + ii
                    plsc.store_scatter(idxl, [pos], gl, mask=sel)
                    plsc.store_scatter(idxg[b], [pos], gl + row * N, mask=sel)
                    return cw + jnp.sum(seli), ct + jnp.sum(eqi)
                lax.fori_loop(0, NB, b2, (jnp.int32(0), jnp.int32(0)))

            for j in range(KB):
                iv = idxl[pl.ds(j * L, L)]
                w = plsc.load_gather(mrow[b], [jnp.right_shift(iv, 2)])
                sh = jnp.left_shift(iv & 3, 3)
                mb[b][pl.ds(j * L, L)] = jnp.right_shift(w, sh) & 1

        def fire_gather(b):
            cp_g[b] = pltpu.async_copy(feats_hbm.at[idxg[b]], rows[b], semg[b])

        def fire_out(r, b):
            row = wid * RPW + r
            c1 = pltpu.async_copy(rows[b], outf_hbm.at[row], semo[b])
            c2 = pltpu.async_copy(mb[b], outm_hbm.at[row], semo[b])
            cp_o[b] = (c1, c2)

        fire_in(0, 0)
        for r in range(RPW):
            b = r % 2
            if r + 1 < RPW:
                fire_in(r + 1, 1 - b)
            for c in cp_in[b]:
                c.wait()
            if r >= 2:
                for c in cp_o[b]:
                    c.wait()
            compute(r, b)
            fire_gather(b)
            if r >= 1:
                cp_g[1 - b].wait()
                fire_out(r - 1, 1 - b)
        lb = (RPW - 1) % 2
        cp_g[lb].wait()
        fire_out(RPW - 1, lb)
        for c in cp_o[1 - lb]:
            c.wait()
        for c in cp_o[lb]:
            c.wait()

    return sc_fn(dists, featsflat, maskw)


def kernel(dists, feats, coord_mask):
    B, N = dists.shape
    D = feats.shape[2]
    K = min(48, N)
    featsflat = feats.reshape(B * N, D)
    maskw = lax.bitcast_convert_type(
        coord_mask.view(jnp.uint8).reshape(B, N // 4, 4), jnp.int32)
    outf, outm = _sc_topk_gather(dists, featsflat, maskw, B, N, D, K)
    return outf, outm != 0
